# Initial kernel scaffold; baseline (speedup 1.0000x reference)
import jax
import jax.numpy as jnp
from jax.experimental import pallas as pl


def kernel(users, items, query_words, word_embedding, entity_embedding, query_proj_w, query_proj_b):
    raise NotImplementedError("write your pallas kernel here")


# speedup vs baseline: 1.8453x; 1.8453x over previous
import functools

import jax
import jax.numpy as jnp
from jax import lax
from jax.experimental import pallas as pl
from jax.experimental.pallas import tpu as pltpu
from jax.experimental.pallas import tpu_sc as plsc

B = 16384
EMB = 64
QLEN = 20
NC = 2
NS = 16
NW = NC * NS
BPW = B // NW
G = 128
UG = BPW // G
WR = BPW * QLEN
WG = WR // G
CB = 32
WGC = CB * QLEN // G
NCHUNK = BPW // CB


def _sc_body(users_hbm, qw_hbm, ent_hbm, word_hbm, user_out, qsum_out,
             uidx, urows, widx, wrows, qbuf, sem):
    wid = lax.axis_index("s") * NC + lax.axis_index("c")
    base = wid * BPW

    pltpu.sync_copy(users_hbm.at[pl.ds(wid * UG, UG)], uidx)
    cps = [
        pltpu.async_copy(ent_hbm.at[uidx.at[j]], urows.at[pl.ds(j * G, G)], sem)
        for j in range(UG)
    ]
    for cp in cps:
        cp.wait()
    pltpu.sync_copy(urows, user_out.at[pl.ds(base, BPW)])

    pltpu.sync_copy(qw_hbm.at[pl.ds(wid * WG, WG)], widx)

    def chunk_body(c, _):
        cps = [
            pltpu.async_copy(word_hbm.at[widx.at[c * WGC + j]],
                             wrows.at[pl.ds(j * G, G)], sem)
            for j in range(WGC)
        ]
        for cp in cps:
            cp.wait()

        def elem_body(i, _):
            r0 = i * QLEN
            for j in range(EMB // 16):
                sl = pl.ds(j * 16, 16)
                acc = wrows[r0, sl]
                for w in range(1, QLEN):
                    acc = acc + wrows[r0 + w, sl]
                qbuf[i, sl] = acc
            return 0

        lax.fori_loop(0, CB, elem_body, 0)
        pltpu.sync_copy(qbuf, qsum_out.at[pl.ds(base + c * CB, CB)])
        return 0

    lax.fori_loop(0, NCHUNK, chunk_body, 0)


_sc_gather = functools.partial(
    pl.kernel,
    out_type=(
        jax.ShapeDtypeStruct((B, EMB), jnp.float32),
        jax.ShapeDtypeStruct((B, EMB), jnp.float32),
    ),
    mesh=plsc.VectorSubcoreMesh(core_axis_name="c", subcore_axis_name="s"),
    compiler_params=pltpu.CompilerParams(use_tc_tiling_on_sc=False),
    scratch_types=[
        pltpu.VMEM((UG, G), jnp.int32),
        pltpu.VMEM((BPW, EMB), jnp.float32),
        pltpu.VMEM((WG, G), jnp.int32),
        pltpu.VMEM((CB * QLEN, EMB), jnp.float32),
        pltpu.VMEM((CB, EMB), jnp.float32),
        pltpu.SemaphoreType.DMA,
    ],
)(_sc_body)


def _tc_body(qsum_ref, user_ref, w_ref, b_ref, out_ref):
    q = qsum_ref[...] * (1.0 / QLEN)
    z = lax.dot_general(q, w_ref[...], (((1,), (1,)), ((), ())),
                        preferred_element_type=jnp.float32)
    z = z + b_ref[...]
    out_ref[...] = 0.5 * jnp.tanh(z) + 0.5 * user_ref[...]


def _tc_call(qsum, user_rows, w, b2d):
    blk = 2048
    return pl.pallas_call(
        _tc_body,
        grid=(B // blk,),
        in_specs=[
            pl.BlockSpec((blk, EMB), lambda i: (i, 0)),
            pl.BlockSpec((blk, EMB), lambda i: (i, 0)),
            pl.BlockSpec((EMB, EMB), lambda i: (0, 0)),
            pl.BlockSpec((1, EMB), lambda i: (0, 0)),
        ],
        out_specs=pl.BlockSpec((blk, EMB), lambda i: (i, 0)),
        out_shape=jax.ShapeDtypeStruct((B, EMB), jnp.float32),
    )(qsum, user_rows, w, b2d)


@jax.jit
def kernel(users, items, query_words, word_embedding, entity_embedding,
           query_proj_w, query_proj_b):
    del items
    users2d = users.reshape(NW * UG, G)
    qw2d = query_words.reshape(NW * WG, G)
    user_rows, qsum = _sc_gather(users2d, qw2d, entity_embedding, word_embedding)
    return _tc_call(qsum, user_rows, query_proj_w,
                    query_proj_b.reshape(1, EMB))


# speedup vs baseline: 1.8875x; 1.0229x over previous
import functools

import jax
import jax.numpy as jnp
from jax import lax
from jax.experimental import pallas as pl
from jax.experimental.pallas import tpu as pltpu
from jax.experimental.pallas import tpu_sc as plsc

B = 16384
EMB = 64
QLEN = 20
NC = 2
NS = 16
NW = NC * NS
BPW = B // NW
CB = 32
NCHUNK = BPW // CB


def _sc_body(users_hbm, qw_hbm, ent_hbm, word_hbm, user_out, qsum_out,
             uidx, widx, ubuf, wrows, qbuf, sem0, sem1):
    wid = lax.axis_index("s") * NC + lax.axis_index("c")
    base = wid * BPW
    sems = (sem0, sem1)

    pltpu.sync_copy(users_hbm.at[pl.ds(base, BPW)], uidx)
    pltpu.sync_copy(qw_hbm.at[pl.ds(base, BPW)], widx)

    def fire(c, p):
        sem = sems[p]
        cps = [pltpu.async_copy(
            ent_hbm.at[uidx.at[pl.ds(c * CB, CB)]], ubuf.at[p], sem)]
        for k in range(CB):
            cps.append(pltpu.async_copy(
                word_hbm.at[widx.at[c * CB + k]],
                wrows.at[p].at[pl.ds(k * QLEN, QLEN)], sem))
        return cps

    live = fire(0, 0)
    for c in range(NCHUNK):
        p = c % 2
        nxt = fire(c + 1, 1 - p) if c + 1 < NCHUNK else []
        for cp in live:
            cp.wait()
        live = nxt

        wr = wrows.at[p]
        qb = qbuf.at[p]

        def elem_body(i, _):
            r0 = i * QLEN
            for j in range(EMB // 16):
                sl = pl.ds(j * 16, 16)
                acc = wr[r0, sl]
                for w in range(1, QLEN):
                    acc = acc + wr[r0 + w, sl]
                qb[i, sl] = acc
            return 0

        lax.fori_loop(0, CB, elem_body, 0)
        pltpu.sync_copy(qb, qsum_out.at[pl.ds(base + c * CB, CB)])
        pltpu.sync_copy(ubuf.at[p], user_out.at[pl.ds(base + c * CB, CB)])


_sc_gather = functools.partial(
    pl.kernel,
    out_type=(
        jax.ShapeDtypeStruct((B, EMB), jnp.float32),
        jax.ShapeDtypeStruct((B, EMB), jnp.float32),
    ),
    mesh=plsc.VectorSubcoreMesh(core_axis_name="c", subcore_axis_name="s"),
    compiler_params=pltpu.CompilerParams(use_tc_tiling_on_sc=False),
    scratch_types=[
        pltpu.VMEM((BPW,), jnp.int32),
        pltpu.VMEM((BPW, QLEN), jnp.int32),
        pltpu.VMEM((2, CB, EMB), jnp.float32),
        pltpu.VMEM((2, CB * QLEN, EMB), jnp.float32),
        pltpu.VMEM((2, CB, EMB), jnp.float32),
        pltpu.SemaphoreType.DMA,
        pltpu.SemaphoreType.DMA,
    ],
)(_sc_body)


def _tc_body(qsum_ref, user_ref, w_ref, b_ref, out_ref):
    q = qsum_ref[...] * (1.0 / QLEN)
    z = lax.dot_general(q, w_ref[...], (((1,), (1,)), ((), ())),
                        preferred_element_type=jnp.float32)
    z = z + b_ref[...]
    out_ref[...] = 0.5 * jnp.tanh(z) + 0.5 * user_ref[...]


def _tc_call(qsum, user_rows, w, b2d):
    blk = 2048
    return pl.pallas_call(
        _tc_body,
        grid=(B // blk,),
        in_specs=[
            pl.BlockSpec((blk, EMB), lambda i: (i, 0)),
            pl.BlockSpec((blk, EMB), lambda i: (i, 0)),
            pl.BlockSpec((EMB, EMB), lambda i: (0, 0)),
            pl.BlockSpec((1, EMB), lambda i: (0, 0)),
        ],
        out_specs=pl.BlockSpec((blk, EMB), lambda i: (i, 0)),
        out_shape=jax.ShapeDtypeStruct((B, EMB), jnp.float32),
    )(qsum, user_rows, w, b2d)


@jax.jit
def kernel(users, items, query_words, word_embedding, entity_embedding,
           query_proj_w, query_proj_b):
    del items
    user_rows, qsum = _sc_gather(users, query_words, entity_embedding,
                                 word_embedding)
    return _tc_call(qsum, user_rows, query_proj_w,
                    query_proj_b.reshape(1, EMB))


# speedup vs baseline: 1.9207x; 1.0175x over previous
import functools

import jax
import jax.numpy as jnp
from jax import lax
from jax.experimental import pallas as pl
from jax.experimental.pallas import tpu as pltpu
from jax.experimental.pallas import tpu_sc as plsc

B = 16384
EMB = 64
QLEN = 20
NC = 2
NS = 16
NW = NC * NS
BPW = B // NW
CB = 32
NCHUNK = BPW // CB


def _sc_body(users_hbm, qw_hbm, ent_hbm, word_hbm, user_out, qsum_out,
             uidx, widx, ubuf, wrows, qbuf, sem0, sem1):
    wid = lax.axis_index("s") * NC + lax.axis_index("c")
    base = wid * BPW
    orow = (wid % (NW // 2)) * BPW
    ocol = (wid // (NW // 2)) * EMB
    sems = (sem0, sem1)

    pltpu.sync_copy(users_hbm.at[pl.ds(base, BPW)], uidx)
    pltpu.sync_copy(qw_hbm.at[pl.ds(base, BPW)], widx)

    def fire(c, p):
        sem = sems[p]
        cps = [pltpu.async_copy(
            ent_hbm.at[uidx.at[pl.ds(c * CB, CB)]], ubuf.at[p], sem)]
        for k in range(CB):
            cps.append(pltpu.async_copy(
                word_hbm.at[widx.at[c * CB + k]],
                wrows.at[p].at[pl.ds(k * QLEN, QLEN)], sem))
        return cps

    live = fire(0, 0)
    for c in range(NCHUNK):
        p = c % 2
        nxt = fire(c + 1, 1 - p) if c + 1 < NCHUNK else []
        for cp in live:
            cp.wait()
        live = nxt

        wr = wrows.at[p]
        qb = qbuf.at[p]

        def elem_body(i, _):
            r0 = i * QLEN
            for j in range(EMB // 16):
                sl = pl.ds(j * 16, 16)
                acc = wr[r0, sl]
                for w in range(1, QLEN):
                    acc = acc + wr[r0 + w, sl]
                qb[i, sl] = acc
            return 0

        lax.fori_loop(0, CB, elem_body, 0)
        pltpu.sync_copy(
            qb, qsum_out.at[pl.ds(orow + c * CB, CB), pl.ds(ocol, EMB)])
        pltpu.sync_copy(
            ubuf.at[p],
            user_out.at[pl.ds(orow + c * CB, CB), pl.ds(ocol, EMB)])


_sc_gather = functools.partial(
    pl.kernel,
    out_type=(
        jax.ShapeDtypeStruct((B // 2, 2 * EMB), jnp.float32),
        jax.ShapeDtypeStruct((B // 2, 2 * EMB), jnp.float32),
    ),
    mesh=plsc.VectorSubcoreMesh(core_axis_name="c", subcore_axis_name="s"),
    compiler_params=pltpu.CompilerParams(use_tc_tiling_on_sc=False),
    scratch_types=[
        pltpu.VMEM((BPW,), jnp.int32),
        pltpu.VMEM((BPW, QLEN), jnp.int32),
        pltpu.VMEM((2, CB, EMB), jnp.float32),
        pltpu.VMEM((2, CB * QLEN, EMB), jnp.float32),
        pltpu.VMEM((2, CB, EMB), jnp.float32),
        pltpu.SemaphoreType.DMA,
        pltpu.SemaphoreType.DMA,
    ],
)(_sc_body)


def _tc_body(qsum_ref, user_ref, w_ref, b_ref, out_ref):
    qp = qsum_ref[...]
    up = user_ref[...]
    q = jnp.concatenate([qp[:, :EMB], qp[:, EMB:]], axis=0) * (1.0 / QLEN)
    u = jnp.concatenate([up[:, :EMB], up[:, EMB:]], axis=0)
    z = lax.dot_general(q, w_ref[...], (((1,), (1,)), ((), ())),
                        preferred_element_type=jnp.float32)
    z = z + b_ref[...]
    out = 0.5 * jnp.tanh(z) + 0.5 * u
    out_ref[...] = out.reshape(2, out.shape[0] // 2, EMB)


def _tc_call(qsum, user_rows, w, b2d):
    blk = 1024
    return pl.pallas_call(
        _tc_body,
        grid=(B // 2 // blk,),
        in_specs=[
            pl.BlockSpec((blk, 2 * EMB), lambda i: (i, 0)),
            pl.BlockSpec((blk, 2 * EMB), lambda i: (i, 0)),
            pl.BlockSpec((EMB, EMB), lambda i: (0, 0)),
            pl.BlockSpec((1, EMB), lambda i: (0, 0)),
        ],
        out_specs=pl.BlockSpec((2, blk, EMB), lambda i: (0, i, 0)),
        out_shape=jax.ShapeDtypeStruct((2, B // 2, EMB), jnp.float32),
    )(qsum, user_rows, w, b2d)


@jax.jit
def kernel(users, items, query_words, word_embedding, entity_embedding,
           query_proj_w, query_proj_b):
    del items
    user_rows, qsum = _sc_gather(users, query_words, entity_embedding,
                                 word_embedding)
    out3d = _tc_call(qsum, user_rows, query_proj_w,
                     query_proj_b.reshape(1, EMB))
    return out3d.reshape(B, EMB)


# speedup vs baseline: 2.0150x; 1.0491x over previous
import functools

import jax
import jax.numpy as jnp
from jax import lax
from jax.experimental import pallas as pl
from jax.experimental.pallas import tpu as pltpu
from jax.experimental.pallas import tpu_sc as plsc

B = 16384
EMB = 64
QLEN = 20
NC = 2
NS = 16
NW = NC * NS
BPW = B // NW
CB = 32
NCHUNK = BPW // CB


def _sc_body(users_hbm, qw_hbm, ent_hbm, word_hbm, user_out, qsum_out,
             uidx, widx, ubuf, wrows, qbuf, sem0, sem1):
    wid = lax.axis_index("s") * NC + lax.axis_index("c")
    base = wid * BPW
    orow = (wid % (NW // 2)) * BPW
    ocol = (wid // (NW // 2)) * EMB
    sems = (sem0, sem1)

    pltpu.sync_copy(users_hbm.at[pl.ds(base, BPW)], uidx)
    pltpu.sync_copy(qw_hbm.at[pl.ds(0, 24), pl.ds(base, BPW)], widx)

    def fire(c, p):
        sem = sems[p]
        cps = [pltpu.async_copy(
            ent_hbm.at[uidx.at[pl.ds(c * CB, CB)]], ubuf.at[p], sem)]
        for j in range(QLEN):
            cps.append(pltpu.async_copy(
                word_hbm.at[widx.at[j, pl.ds(c * CB, CB)]],
                wrows.at[p].at[j], sem))
        return cps

    live = fire(0, 0)
    for c in range(NCHUNK):
        p = c % 2
        nxt = fire(c + 1, 1 - p) if c + 1 < NCHUNK else []
        for cp in live:
            cp.wait()
        live = nxt

        wr = wrows.at[p]
        qb = qbuf.at[p]

        def elem_body(i, _):
            for j in range(EMB // 16):
                sl = pl.ds(j * 16, 16)
                acc = wr[0, i, sl]
                for w in range(1, QLEN):
                    acc = acc + wr[w, i, sl]
                qb[i, sl] = acc
            return 0

        lax.fori_loop(0, CB, elem_body, 0)
        pltpu.sync_copy(
            qb, qsum_out.at[pl.ds(orow + c * CB, CB), pl.ds(ocol, EMB)])
        pltpu.sync_copy(
            ubuf.at[p],
            user_out.at[pl.ds(orow + c * CB, CB), pl.ds(ocol, EMB)])


_sc_gather = functools.partial(
    pl.kernel,
    out_type=(
        jax.ShapeDtypeStruct((B // 2, 2 * EMB), jnp.float32),
        jax.ShapeDtypeStruct((B // 2, 2 * EMB), jnp.float32),
    ),
    mesh=plsc.VectorSubcoreMesh(core_axis_name="c", subcore_axis_name="s"),
    compiler_params=pltpu.CompilerParams(use_tc_tiling_on_sc=False),
    scratch_types=[
        pltpu.VMEM((BPW,), jnp.int32),
        pltpu.VMEM((24, BPW), jnp.int32),
        pltpu.VMEM((2, CB, EMB), jnp.float32),
        pltpu.VMEM((2, QLEN, CB, EMB), jnp.float32),
        pltpu.VMEM((2, CB, EMB), jnp.float32),
        pltpu.SemaphoreType.DMA,
        pltpu.SemaphoreType.DMA,
    ],
)(_sc_body)


def _tc_body(qsum_ref, user_ref, w_ref, b_ref, out_ref):
    qp = qsum_ref[...]
    up = user_ref[...]
    q = jnp.concatenate([qp[:, :EMB], qp[:, EMB:]], axis=0) * (1.0 / QLEN)
    u = jnp.concatenate([up[:, :EMB], up[:, EMB:]], axis=0)
    z = lax.dot_general(q, w_ref[...], (((1,), (1,)), ((), ())),
                        preferred_element_type=jnp.float32)
    z = z + b_ref[...]
    out = 0.5 * jnp.tanh(z) + 0.5 * u
    out_ref[...] = out.reshape(2, out.shape[0] // 2, EMB)


def _tc_call(qsum, user_rows, w, b2d):
    blk = 1024
    return pl.pallas_call(
        _tc_body,
        grid=(B // 2 // blk,),
        in_specs=[
            pl.BlockSpec((blk, 2 * EMB), lambda i: (i, 0)),
            pl.BlockSpec((blk, 2 * EMB), lambda i: (i, 0)),
            pl.BlockSpec((EMB, EMB), lambda i: (0, 0)),
            pl.BlockSpec((1, EMB), lambda i: (0, 0)),
        ],
        out_specs=pl.BlockSpec((2, blk, EMB), lambda i: (0, i, 0)),
        out_shape=jax.ShapeDtypeStruct((2, B // 2, EMB), jnp.float32),
    )(qsum, user_rows, w, b2d)


@jax.jit
def kernel(users, items, query_words, word_embedding, entity_embedding,
           query_proj_w, query_proj_b):
    del items
    qw_t = jnp.pad(query_words.T, ((0, 24 - QLEN), (0, 0)))
    user_rows, qsum = _sc_gather(users, qw_t, entity_embedding,
                                 word_embedding)
    out3d = _tc_call(qsum, user_rows, query_proj_w,
                     query_proj_b.reshape(1, EMB))
    return out3d.reshape(B, EMB)
